# Initial kernel scaffold; baseline (speedup 1.0000x reference)
import jax
import jax.numpy as jnp
from jax.experimental import pallas as pl


def kernel(x_embed, prompt):
    raise NotImplementedError("write your pallas kernel here")


# speedup vs baseline: 1.1517x; 1.1517x over previous
import jax
import jax.numpy as jnp
from jax.experimental import pallas as pl
from jax.experimental.pallas import tpu as pltpu

_B, _S, _D = 4, 2048, 768
_P = 1000
_TOPK = 4
_SCH = 64
_PCH = 40


def _body(x_ref, prompt_ref, out_ref, simcol_ref, idx_ref, rsum_ref):
    b = pl.program_id(0)

    acc = jnp.zeros((1, _D), jnp.float32)
    for i in range(_S // _SCH):
        blk = x_ref[0, i * _SCH:(i + 1) * _SCH, :]
        out_ref[0, _TOPK + i * _SCH:_TOPK + (i + 1) * _SCH, :] = blk
        acc = acc + jnp.sum(blk, axis=0, keepdims=True)
    mean = acc * (1.0 / _S)
    xr = jax.lax.rsqrt(jnp.maximum(jnp.sum(mean * mean), 1e-12))

    for j in range(_P // _PCH):
        pc = prompt_ref[j * _PCH:(j + 1) * _PCH, :]
        d = jnp.sum(pc * mean, axis=1, keepdims=True)
        pr = jax.lax.rsqrt(
            jnp.maximum(jnp.sum(pc * pc, axis=1, keepdims=True), 1e-12))
        simcol_ref[0, j * _PCH:(j + 1) * _PCH, :] = d * (pr * xr)

    iota = jax.lax.broadcasted_iota(jnp.int32, (_P, 1), 0)
    cur = simcol_ref[0, :, :]
    tops = jnp.float32(0.0)
    for k in range(_TOPK):
        m = jnp.max(cur)
        ak = jnp.min(jnp.where(cur == m, iota, jnp.int32(_P)))
        idx_ref[0, 0, k] = ak
        out_ref[0, k:k + 1, :] = prompt_ref[pl.ds(ak, 1), :]
        tops = tops + m
        cur = jnp.where(iota == ak, -jnp.inf, cur)

    prev = jnp.where(b == 0, 0.0, rsum_ref[0, 0])
    rsum_ref[0, 0] = prev + tops * (1.0 / _B)


@jax.jit
def kernel(x_embed, prompt):
    out, simcol, idx, rsum = pl.pallas_call(
        _body,
        grid=(_B,),
        in_specs=[
            pl.BlockSpec((1, _S, _D), lambda b: (b, 0, 0)),
            pl.BlockSpec((_P, _D), lambda b: (0, 0)),
        ],
        out_specs=[
            pl.BlockSpec((1, _S + _TOPK, _D), lambda b: (b, 0, 0)),
            pl.BlockSpec((1, _P, 1), lambda b: (b, 0, 0)),
            pl.BlockSpec((1, 1, _TOPK), lambda b: (b, 0, 0),
                         memory_space=pltpu.SMEM),
            pl.BlockSpec((1, 1), lambda b: (0, 0),
                         memory_space=pltpu.SMEM),
        ],
        out_shape=[
            jax.ShapeDtypeStruct((_B, _S + _TOPK, _D), jnp.float32),
            jax.ShapeDtypeStruct((_B, _P, 1), jnp.float32),
            jax.ShapeDtypeStruct((_B, 1, _TOPK), jnp.int32),
            jax.ShapeDtypeStruct((1, 1), jnp.float32),
        ],
        compiler_params=pltpu.CompilerParams(
            dimension_semantics=("arbitrary",)),
    )(x_embed, prompt)
    return out, rsum[0, 0], simcol.reshape(_B, _P), idx.reshape(_B, _TOPK)
